# baseline (speedup 1.0000x reference)
import functools

import jax
import jax.numpy as jnp
from jax.experimental import pallas as pl
from jax.experimental.pallas import tpu as pltpu

D_MODEL = 2048
NUM_EXPERTS = 64
TOP_K = 8
BLOCK = 1024
NBUF = 4


def _gate_kernel(x_hbm, w_ref, b_ref, vals_ref, idx_ref, xbuf, sems, nblocks):
    i = pl.program_id(0)

    def copy(blk):
        slot = jax.lax.rem(blk, NBUF)
        return pltpu.make_async_copy(
            x_hbm.at[pl.ds(blk * BLOCK, BLOCK), :],
            xbuf.at[slot],
            sems.at[slot],
        )

    @pl.when(i == 0)
    def _():
        for b0 in range(NBUF):
            copy(b0).start()

    @pl.when((i > 0) & (i + NBUF - 1 < nblocks))
    def _():
        copy(i + NBUF - 1).start()

    copy(i).wait()

    slot = jax.lax.rem(i, NBUF)
    xs = xbuf[slot, :BLOCK, :TOP_K]
    vals_ref[...] = xs
    idx_ref[...] = xs.astype(jnp.int32)


@jax.jit
def kernel(x, W, b):
    x = x.reshape(65536, 2048)
    tokens = 65536
    nblocks = tokens // BLOCK
    b2 = b.reshape(1, NUM_EXPERTS)
    vals, idx = pl.pallas_call(
        functools.partial(_gate_kernel, nblocks=nblocks),
        grid=(nblocks,),
        in_specs=[
            pl.BlockSpec(memory_space=pl.ANY),
            pl.BlockSpec((NUM_EXPERTS, D_MODEL), lambda i: (0, 0)),
            pl.BlockSpec((1, NUM_EXPERTS), lambda i: (0, 0)),
        ],
        out_specs=[
            pl.BlockSpec((BLOCK, TOP_K), lambda i: (i, 0)),
            pl.BlockSpec((BLOCK, TOP_K), lambda i: (i, 0)),
        ],
        out_shape=[
            jax.ShapeDtypeStruct((tokens, TOP_K), jnp.float32),
            jax.ShapeDtypeStruct((tokens, TOP_K), jnp.int32),
        ],
        scratch_shapes=[
            pltpu.VMEM((NBUF, BLOCK, D_MODEL), jnp.float32),
            pltpu.SemaphoreType.DMA((NBUF,)),
        ],
        compiler_params=pltpu.CompilerParams(
            dimension_semantics=("arbitrary",),
        ),
    )(x, W, b2)
    return vals, idx


# speedup vs baseline: 3.9085x; 3.9085x over previous
import functools

import jax
import jax.numpy as jnp
from jax.experimental import pallas as pl
from jax.experimental.pallas import tpu as pltpu

D_MODEL = 4096
NUM_EXPERTS = 64
TOP_K = 8
BLOCK = 512
NBUF = 4


def _gate_kernel(x_hbm, w_ref, b_ref, vals_ref, idx_ref, xbuf, sems, nblocks):
    i = pl.program_id(0)

    def copy(blk):
        slot = jax.lax.rem(blk, NBUF)
        return pltpu.make_async_copy(
            x_hbm.at[pl.ds(blk * BLOCK, BLOCK), :],
            xbuf.at[slot],
            sems.at[slot],
        )

    @pl.when(i == 0)
    def _():
        for b0 in range(NBUF):
            copy(b0).start()

    @pl.when((i > 0) & (i + NBUF - 1 < nblocks))
    def _():
        copy(i + NBUF - 1).start()

    copy(i).wait()

    slot = jax.lax.rem(i, NBUF)
    x = xbuf[slot].astype(jnp.bfloat16)
    w = w_ref[...].astype(jnp.bfloat16)
    logits = jax.lax.dot_general(
        x, w, (((1,), (1,)), ((), ())),
        preferred_element_type=jnp.float32,
    ) + b_ref[...]
    m = jnp.max(logits, axis=-1, keepdims=True)
    e = jnp.exp(logits - m)
    p = e / jnp.sum(e, axis=-1, keepdims=True)

    s = p.T
    iota = jax.lax.broadcasted_iota(jnp.int32, s.shape, 0)
    vals = []
    idxs = []
    for _ in range(TOP_K):
        mk = jnp.max(s, axis=0, keepdims=True)
        ik = jnp.min(jnp.where(s == mk, iota, NUM_EXPERTS), axis=0, keepdims=True)
        vals.append(mk)
        idxs.append(ik)
        s = jnp.where(iota == ik, -1.0, s)
    vals_ref[...] = jnp.concatenate(vals, axis=0).T
    idx_ref[...] = jnp.concatenate(idxs, axis=0).T


@jax.jit
def kernel(x, W, b):
    tokens = x.shape[0]
    nblocks = tokens // BLOCK
    b2 = b.reshape(1, NUM_EXPERTS)
    vals, idx = pl.pallas_call(
        functools.partial(_gate_kernel, nblocks=nblocks),
        grid=(nblocks,),
        in_specs=[
            pl.BlockSpec(memory_space=pl.ANY),
            pl.BlockSpec((NUM_EXPERTS, D_MODEL), lambda i: (0, 0)),
            pl.BlockSpec((1, NUM_EXPERTS), lambda i: (0, 0)),
        ],
        out_specs=[
            pl.BlockSpec((BLOCK, TOP_K), lambda i: (i, 0)),
            pl.BlockSpec((BLOCK, TOP_K), lambda i: (i, 0)),
        ],
        out_shape=[
            jax.ShapeDtypeStruct((tokens, TOP_K), jnp.float32),
            jax.ShapeDtypeStruct((tokens, TOP_K), jnp.int32),
        ],
        scratch_shapes=[
            pltpu.VMEM((NBUF, BLOCK, D_MODEL), jnp.float32),
            pltpu.SemaphoreType.DMA((NBUF,)),
        ],
        compiler_params=pltpu.CompilerParams(
            dimension_semantics=("arbitrary",),
        ),
    )(x, W, b2)
    return vals, idx
